# baseline (speedup 1.0000x reference)
import jax
import jax.numpy as jnp
from jax.experimental import pallas as pl
from jax.experimental.pallas import tpu as pltpu

_N = 8192
_D = 64
_CUTOFF = 0.5
_MU = 1.0
_SIGMA = 0.5
_C2 = -1.4426950408889634 / (2.0 * _SIGMA * _SIGMA)

_BR = 256
_NBUF = 3
_NOBUF = 2
_NBLK = _N // _BR


def _interact_kernel(z_hbm, w_ref, b_ref, dist_hbm, out_hbm,
                     buf, zbuf, msg, obuf, in_sems, z_sem, out_sems):
    def copy_in(blk, slot):
        return pltpu.make_async_copy(
            dist_hbm.at[pl.ds(blk * _BR, _BR), :],
            buf.at[slot],
            in_sems.at[slot],
        )

    def copy_out(blk, slot):
        return pltpu.make_async_copy(
            obuf.at[slot],
            out_hbm.at[pl.ds(blk * _BR, _BR), :],
            out_sems.at[slot],
        )

    z_copy = pltpu.make_async_copy(z_hbm, zbuf, z_sem)
    z_copy.start()
    for s in range(_NBUF):
        copy_in(s, s).start()
    z_copy.wait()

    msg[...] = (
        jnp.dot(zbuf[...], w_ref[...], preferred_element_type=jnp.float32)
        + b_ref[...]
    ).astype(jnp.bfloat16)

    def body(blk, carry):
        slot = jax.lax.rem(blk, _NBUF)
        oslot = jax.lax.rem(blk, _NOBUF)
        copy_in(blk, slot).wait()
        r = buf[slot]
        u = 1.0 / r
        t = u - 1.0 / _MU
        w = jnp.where(r < _CUTOFF, jnp.exp2(_C2 * (t * t)), 0.0).astype(jnp.bfloat16)

        @pl.when(blk >= _NOBUF)
        def _drain():
            copy_out(blk - _NOBUF, oslot).wait()

        obuf[oslot] = jnp.dot(w, msg[...], preferred_element_type=jnp.float32)
        copy_out(blk, oslot).start()

        @pl.when(blk + _NBUF < _NBLK)
        def _prefetch():
            copy_in(blk + _NBUF, slot).start()

        return carry

    jax.lax.fori_loop(0, _NBLK, body, 0)
    for t in range(_NOBUF):
        blk = _NBLK - _NOBUF + t
        copy_out(blk, blk % _NOBUF).wait()


def kernel(z, dist_matrix, W, B):
    out = pl.pallas_call(
        _interact_kernel,
        in_specs=[
            pl.BlockSpec(memory_space=pltpu.HBM),
            pl.BlockSpec((_D, _D), lambda: (0, 0)),
            pl.BlockSpec((1, _D), lambda: (0, 0)),
            pl.BlockSpec(memory_space=pltpu.HBM),
        ],
        out_specs=pl.BlockSpec(memory_space=pltpu.HBM),
        out_shape=jax.ShapeDtypeStruct((_N, _D), jnp.float32),
        scratch_shapes=[
            pltpu.VMEM((_NBUF, _BR, _N), jnp.float32),
            pltpu.VMEM((_N, _D), jnp.float32),
            pltpu.VMEM((_N, _D), jnp.bfloat16),
            pltpu.VMEM((_NOBUF, _BR, _D), jnp.float32),
            pltpu.SemaphoreType.DMA((_NBUF,)),
            pltpu.SemaphoreType.DMA,
            pltpu.SemaphoreType.DMA((_NOBUF,)),
        ],
    )(z, W, B.reshape(1, _D), dist_matrix)
    return out


# speedup vs baseline: 1.5155x; 1.5155x over previous
import jax
import jax.numpy as jnp
from jax.experimental import pallas as pl
from jax.experimental.pallas import tpu as pltpu

_N = 8192
_D = 64
_CUTOFF = 0.5
_MU = 1.0
_SIGMA = 0.5
_C2 = -1.4426950408889634 / (2.0 * _SIGMA * _SIGMA)

_BR = 256
_NBUF = 3
_NBLK = _N // _BR


def _interact_kernel(z_hbm, w_ref, b_ref, dist_hbm, out_ref, buf, zbuf, msg, sems, z_sem):
    def copy_in(blk, slot):
        return pltpu.make_async_copy(
            dist_hbm.at[pl.ds(blk * _BR, _BR), :],
            buf.at[slot],
            sems.at[slot],
        )

    z_copy = pltpu.make_async_copy(z_hbm, zbuf, z_sem)
    z_copy.start()
    for s in range(_NBUF):
        copy_in(s, s).start()
    z_copy.wait()

    msg[...] = (
        jnp.dot(zbuf[...], w_ref[...], preferred_element_type=jnp.float32)
        + b_ref[...]
    ).astype(jnp.bfloat16)

    def body(blk, carry):
        slot = jax.lax.rem(blk, _NBUF)
        copy_in(blk, slot).wait()
        r = buf[slot]
        u = 1.0 / r
        t = u - 1.0 / _MU
        w = jnp.where(r < _CUTOFF, jnp.exp2(_C2 * (t * t)), 0.0).astype(jnp.bfloat16)
        out_ref[pl.ds(blk * _BR, _BR), :] = jnp.dot(
            w, msg[...], preferred_element_type=jnp.float32
        )

        @pl.when(blk + _NBUF < _NBLK)
        def _prefetch():
            copy_in(blk + _NBUF, slot).start()

        return carry

    jax.lax.fori_loop(0, _NBLK, body, 0)


def kernel(z, dist_matrix, W, B):
    out = pl.pallas_call(
        _interact_kernel,
        in_specs=[
            pl.BlockSpec(memory_space=pltpu.HBM),
            pl.BlockSpec((_D, _D), lambda: (0, 0)),
            pl.BlockSpec((1, _D), lambda: (0, 0)),
            pl.BlockSpec(memory_space=pltpu.HBM),
        ],
        out_specs=pl.BlockSpec((_N, _D), lambda: (0, 0)),
        out_shape=jax.ShapeDtypeStruct((_N, _D), jnp.float32),
        scratch_shapes=[
            pltpu.VMEM((_NBUF, _BR, _N), jnp.float32),
            pltpu.VMEM((_N, _D), jnp.float32),
            pltpu.VMEM((_N, _D), jnp.bfloat16),
            pltpu.SemaphoreType.DMA((_NBUF,)),
            pltpu.SemaphoreType.DMA,
        ],
    )(z, W, B.reshape(1, _D), dist_matrix)
    return out
